# baseline (speedup 1.0000x reference)
import functools

import jax
import jax.numpy as jnp
from jax import lax
from jax.experimental import pallas as pl
from jax.experimental.pallas import tpu as pltpu
from jax.experimental.pallas import tpu_sc as plsc

NC = 2
NS = 16
NW = NC * NS
CH = 128


def _mesh():
    return plsc.VectorSubcoreMesh(core_axis_name="c", subcore_axis_name="s")


def _sc_degree(dstp, n, acc_n, cpt):
    del n
    @functools.partial(
        pl.kernel,
        out_type=jax.ShapeDtypeStruct((NC, acc_n), jnp.float32),
        mesh=_mesh(),
        scratch_types=[
            pltpu.VMEM((cpt, CH), jnp.int32),
            pltpu.VMEM((CH,), jnp.float32),
            pltpu.VMEM((acc_n // NS,), jnp.float32),
            pltpu.VMEM_SHARED((acc_n,), jnp.float32),
        ],
    )
    def k(dst_hbm, out_hbm, idx_v, ones_v, z_v, deg_sh):
        c = lax.axis_index("c")
        s = lax.axis_index("s")
        w = s * NC + c
        zslice = acc_n // NS

        def fill_ones(i, _):
            ones_v[pl.ds(i * 16, 16)] = jnp.ones((16,), jnp.float32)
            return 0

        lax.fori_loop(0, CH // 16, fill_ones, 0)

        def fill_zeros(i, _):
            z_v[pl.ds(i * 16, 16)] = jnp.zeros((16,), jnp.float32)
            return 0

        lax.fori_loop(0, zslice // 16, fill_zeros, 0)

        pltpu.sync_copy(z_v, deg_sh.at[pl.ds(s * zslice, zslice)])
        pltpu.sync_copy(dst_hbm.at[w], idx_v)
        plsc.subcore_barrier()

        def body(j, _):
            pltpu.sync_copy(ones_v, deg_sh.at[idx_v.at[j]], add=True)
            return 0

        lax.fori_loop(0, cpt, body, 0)
        plsc.subcore_barrier()

        @pl.when(s == 0)
        def _():
            pltpu.sync_copy(deg_sh.at[pl.ds(0, acc_n)], out_hbm.at[c])

    return k(dstp)


def _sc_aggregate(ts, srcp, dstA, dstB, acc_n, half_n, half_acc, cpt):
    d = ts.shape[1]

    @functools.partial(
        pl.kernel,
        out_type=jax.ShapeDtypeStruct((NC, acc_n, d), jnp.float32),
        mesh=_mesh(),
        scratch_types=[
            pltpu.VMEM((cpt, CH), jnp.int32),
            pltpu.VMEM((cpt, CH), jnp.int32),
            pltpu.VMEM((2, CH, d), jnp.float32),
            pltpu.VMEM((CH, d), jnp.float32),
            pltpu.VMEM_SHARED((half_acc, d), jnp.float32),
            pltpu.SemaphoreType.DMA((2,)),
        ],
    )
    def k(ts_hbm, src_hbm, dstA_hbm, dstB_hbm, out_hbm,
          si_v, di_v, rows_v, z_v, acc_sh, gsem):
        c = lax.axis_index("c")
        s = lax.axis_index("s")
        w = s * NC + c
        zrows_per_tile = half_acc // NS
        orows_per_tile = half_n // NS

        def zrow(r, _):
            def zcol(kk, _):
                z_v[r, pl.ds(kk * 16, 16)] = jnp.zeros((16,), jnp.float32)
                return 0

            lax.fori_loop(0, d // 16, zcol, 0)
            return 0

        lax.fori_loop(0, CH, zrow, 0)
        pltpu.sync_copy(src_hbm.at[w], si_v)

        for half, dst_hbm in ((0, dstA_hbm), (1, dstB_hbm)):
            def zblk(i, _):
                pltpu.sync_copy(z_v, acc_sh.at[pl.ds(s * zrows_per_tile + i * CH, CH)])
                return 0

            lax.fori_loop(0, zrows_per_tile // CH, zblk, 0)
            pltpu.sync_copy(dst_hbm.at[w], di_v)
            plsc.subcore_barrier()

            pltpu.make_async_copy(
                ts_hbm.at[si_v.at[0]], rows_v.at[0], gsem.at[0]
            ).start()

            def body(i, _):
                j0 = i * 2
                pltpu.make_async_copy(
                    ts_hbm.at[si_v.at[j0 + 1]], rows_v.at[1], gsem.at[1]
                ).start()
                pltpu.make_async_copy(
                    ts_hbm.at[si_v.at[j0]], rows_v.at[0], gsem.at[0]
                ).wait()
                pltpu.sync_copy(rows_v.at[0], acc_sh.at[di_v.at[j0]], add=True)

                @pl.when(j0 + 2 < cpt)
                def _():
                    pltpu.make_async_copy(
                        ts_hbm.at[si_v.at[j0 + 2]], rows_v.at[0], gsem.at[0]
                    ).start()

                pltpu.make_async_copy(
                    ts_hbm.at[si_v.at[j0 + 1]], rows_v.at[1], gsem.at[1]
                ).wait()
                pltpu.sync_copy(rows_v.at[1], acc_sh.at[di_v.at[j0 + 1]], add=True)
                return 0

            lax.fori_loop(0, cpt // 2, body, 0)
            plsc.subcore_barrier()
            pltpu.sync_copy(
                acc_sh.at[pl.ds(s * orows_per_tile, orows_per_tile)],
                out_hbm.at[c, pl.ds(half * half_n + s * orows_per_tile,
                                    orows_per_tile)],
            )
            plsc.subcore_barrier()

    return k(ts, srcp, dstA, dstB)


def _tc_prep(deg2, dstp, n, half_n, half_acc):
    acc_n = deg2.shape[1]
    nw, cpt, ch = dstp.shape
    flat = (nw * cpt, ch)
    dstf = dstp.reshape(flat)

    def body(deg_ref, dst_ref, dinv_ref, dstA_ref, dstB_ref):
        dinv_ref[...] = lax.rsqrt(deg_ref[0:1, :] + deg_ref[1:2, :] + 1.0)
        dst = dst_ref[...]
        pos = (jax.lax.broadcasted_iota(jnp.int32, flat, 0) * ch
               + jax.lax.broadcasted_iota(jnp.int32, flat, 1))
        dump = half_n + lax.rem(pos, half_acc - half_n)
        dstA_ref[...] = jnp.where(dst < half_n, dst, dump)
        inB = jnp.logical_and(dst >= half_n, dst < n)
        dstB_ref[...] = jnp.where(inB, dst - half_n, dump)

    dinv, dstA, dstB = pl.pallas_call(
        body,
        out_shape=[
            jax.ShapeDtypeStruct((1, acc_n), jnp.float32),
            jax.ShapeDtypeStruct(flat, jnp.int32),
            jax.ShapeDtypeStruct(flat, jnp.int32),
        ],
    )(deg2, dstf)
    return dinv, dstA.reshape(nw, cpt, ch), dstB.reshape(nw, cpt, ch)


def _tc_first(x, W1, dinv, blk):
    n, d = x.shape

    def body(x_ref, w_ref, dv_ref, out_ref):
        out_ref[...] = (
            jnp.dot(x_ref[...], w_ref[...], preferred_element_type=jnp.float32)
            * dv_ref[...]
        )

    return pl.pallas_call(
        body,
        grid=(n // blk,),
        in_specs=[
            pl.BlockSpec((blk, d), lambda i: (i, 0)),
            pl.BlockSpec((d, d), lambda i: (0, 0)),
            pl.BlockSpec((blk, 1), lambda i: (i, 0)),
        ],
        out_specs=pl.BlockSpec((blk, d), lambda i: (i, 0)),
        out_shape=jax.ShapeDtypeStruct((n, d), jnp.float32),
    )(x, W1, dinv)


def _tc_layer(p, ts, dinv, b, W, blk):
    n, d = ts.shape

    def body(p_ref, ts_ref, dv_ref, b_ref, w_ref, out_ref):
        h = (p_ref[0] + p_ref[1] + ts_ref[...]) * dv_ref[...] + b_ref[...]
        h = jnp.maximum(h, 0.0)
        out_ref[...] = (
            jnp.dot(h, w_ref[...], preferred_element_type=jnp.float32) * dv_ref[...]
        )

    return pl.pallas_call(
        body,
        grid=(n // blk,),
        in_specs=[
            pl.BlockSpec((2, blk, d), lambda i: (0, i, 0)),
            pl.BlockSpec((blk, d), lambda i: (i, 0)),
            pl.BlockSpec((blk, 1), lambda i: (i, 0)),
            pl.BlockSpec((1, d), lambda i: (0, 0)),
            pl.BlockSpec((d, d), lambda i: (0, 0)),
        ],
        out_specs=pl.BlockSpec((blk, d), lambda i: (i, 0)),
        out_shape=jax.ShapeDtypeStruct((n, d), jnp.float32),
    )(p, ts, dinv, b, W)


def _tc_head(p, ts, dinv, b3, L1w, L1b, L2w, L2b, blk):
    n, d = ts.shape
    g_steps = n // blk

    def body(p_ref, ts_ref, dv_ref, b_ref, l1w_ref, l1b_ref, l2w_ref, l2b_ref,
             out_ref, acc_ref):
        i = pl.program_id(0)
        h = (p_ref[0] + p_ref[1] + ts_ref[...]) * dv_ref[...] + b_ref[...]
        h = jnp.maximum(h, 0.0)
        bsum = jnp.sum(h, axis=0, keepdims=True)

        @pl.when(i == 0)
        def _():
            acc_ref[...] = bsum

        @pl.when(i > 0)
        def _():
            acc_ref[...] = acc_ref[...] + bsum

        @pl.when(i == g_steps - 1)
        def _():
            g = acc_ref[...] * (1.0 / n)
            z = jnp.dot(g, l1w_ref[...], preferred_element_type=jnp.float32)
            z = jnp.maximum(z + l1b_ref[...], 0.0)
            o = jnp.dot(z, l2w_ref[...], preferred_element_type=jnp.float32)
            out_ref[...] = jax.nn.sigmoid(o + l2b_ref[...])

    return pl.pallas_call(
        body,
        grid=(g_steps,),
        in_specs=[
            pl.BlockSpec((2, blk, d), lambda i: (0, i, 0)),
            pl.BlockSpec((blk, d), lambda i: (i, 0)),
            pl.BlockSpec((blk, 1), lambda i: (i, 0)),
            pl.BlockSpec((1, d), lambda i: (0, 0)),
            pl.BlockSpec((d, d), lambda i: (0, 0)),
            pl.BlockSpec((1, d), lambda i: (0, 0)),
            pl.BlockSpec((d, 1), lambda i: (0, 0)),
            pl.BlockSpec((1, 1), lambda i: (0, 0)),
        ],
        out_specs=pl.BlockSpec((1, 1), lambda i: (0, 0)),
        out_shape=jax.ShapeDtypeStruct((1, 1), jnp.float32),
        scratch_shapes=[pltpu.VMEM((1, d), jnp.float32)],
    )(p, ts, dinv, b3, L1w, L1b, L2w, L2b)


def kernel(x, edge_index, W1, b1, W2, b2, W3, b3, L1w, L1b, L2w, L2b):
    n, d = x.shape
    e = edge_index.shape[1]
    blk = 2000

    ept = ((e + NW * CH - 1) // (NW * CH)) * CH
    if (ept // CH) % 2:
        ept += CH
    cpt = ept // CH
    epad = NW * ept
    acc_n = ((n + NS * CH) // (NS * CH)) * NS * CH

    src = edge_index[0]
    dst = edge_index[1]
    pad = epad - e
    srcp = jnp.concatenate([src, jnp.zeros((pad,), jnp.int32)]).reshape(NW, cpt, CH)
    dstp = jnp.concatenate([dst, jnp.full((pad,), n, jnp.int32)]).reshape(NW, cpt, CH)

    half_n = acc_n // 2
    half_acc = ((half_n + 1 + NS * CH - 1) // (NS * CH)) * (NS * CH)

    deg2 = _sc_degree(dstp, n, acc_n, cpt)
    dinv, dstA, dstB = _tc_prep(deg2, dstp, n, half_n, half_acc)
    dinv = dinv.reshape(acc_n, 1)

    ts = _tc_first(x, W1, dinv, blk)
    p = _sc_aggregate(ts, srcp, dstA, dstB, acc_n, half_n, half_acc, cpt)
    ts = _tc_layer(p, ts, dinv, b1.reshape(1, d), W2, blk)
    p = _sc_aggregate(ts, srcp, dstA, dstB, acc_n, half_n, half_acc, cpt)
    ts = _tc_layer(p, ts, dinv, b2.reshape(1, d), W3, blk)
    p = _sc_aggregate(ts, srcp, dstA, dstB, acc_n, half_n, half_acc, cpt)
    out = _tc_head(p, ts, dinv, b3.reshape(1, d), L1w, L1b.reshape(1, d),
                   L2w, L2b.reshape(1, 1), blk)
    return out.reshape(1)


# speedup vs baseline: 1.9712x; 1.9712x over previous
import functools

import jax
import jax.numpy as jnp
from jax import lax
from jax.experimental import pallas as pl
from jax.experimental.pallas import tpu as pltpu
from jax.experimental.pallas import tpu_sc as plsc

NC = 2
NS = 16
NW = NC * NS
CH = 128


def _mesh():
    return plsc.VectorSubcoreMesh(core_axis_name="c", subcore_axis_name="s")


def _sc_degree(dstp, n, acc_n, cpt):
    del n
    @functools.partial(
        pl.kernel,
        out_type=jax.ShapeDtypeStruct((NC, acc_n), jnp.float32),
        mesh=_mesh(),
        scratch_types=[
            pltpu.VMEM((cpt, CH), jnp.int32),
            pltpu.VMEM((CH,), jnp.float32),
            pltpu.VMEM((acc_n // NS,), jnp.float32),
            pltpu.VMEM_SHARED((acc_n,), jnp.float32),
        ],
    )
    def k(dst_hbm, out_hbm, idx_v, ones_v, z_v, deg_sh):
        c = lax.axis_index("c")
        s = lax.axis_index("s")
        w = s * NC + c
        zslice = acc_n // NS

        def fill_ones(i, _):
            ones_v[pl.ds(i * 16, 16)] = jnp.ones((16,), jnp.float32)
            return 0

        lax.fori_loop(0, CH // 16, fill_ones, 0)

        def fill_zeros(i, _):
            z_v[pl.ds(i * 16, 16)] = jnp.zeros((16,), jnp.float32)
            return 0

        lax.fori_loop(0, zslice // 16, fill_zeros, 0)

        pltpu.sync_copy(z_v, deg_sh.at[pl.ds(s * zslice, zslice)])
        pltpu.sync_copy(dst_hbm.at[w], idx_v)
        plsc.subcore_barrier()

        def body(j, _):
            pltpu.sync_copy(ones_v, deg_sh.at[idx_v.at[j]], add=True)
            return 0

        lax.fori_loop(0, cpt, body, 0)
        plsc.subcore_barrier()

        @pl.when(s == 0)
        def _():
            pltpu.sync_copy(deg_sh.at[pl.ds(0, acc_n)], out_hbm.at[c])

    return k(dstp)


def _sc_partition(srcp, dstA, dstB, half_n, half_acc, cpt):
    dumpreg = half_acc - half_n
    ept = cpt * CH
    slack = 4 * CH + 2 * 16

    @functools.partial(
        pl.kernel,
        out_type=(
            jax.ShapeDtypeStruct((2, NW, ept), jnp.int32),
            jax.ShapeDtypeStruct((2, NW, ept), jnp.int32),
            jax.ShapeDtypeStruct((NW, 2, CH), jnp.int32),
        ),
        mesh=_mesh(),
        scratch_types=[
            pltpu.VMEM((cpt, CH), jnp.int32),
            pltpu.VMEM((cpt, CH), jnp.int32),
            pltpu.VMEM((cpt, CH), jnp.int32),
            pltpu.VMEM((ept + slack,), jnp.int32),
            pltpu.VMEM((ept + slack,), jnp.int32),
            pltpu.VMEM((ept + slack,), jnp.int32),
            pltpu.VMEM((ept + slack,), jnp.int32),
            pltpu.VMEM((2, CH), jnp.int32),
        ],
    )
    def k(src_hbm, dA_hbm, dB_hbm, srcL_hbm, dstL_hbm, cnt_hbm,
          si_v, dA_v, dB_v, sA_v, tA_v, sB_v, tB_v, cnt_v):
        c = lax.axis_index("c")
        s = lax.axis_index("s")
        w = s * NC + c
        pltpu.sync_copy(src_hbm.at[w], si_v)
        pltpu.sync_copy(dA_hbm.at[w], dA_v)
        pltpu.sync_copy(dB_hbm.at[w], dB_v)
        lanes = lax.iota(jnp.int32, 16)

        def grp(g, carry):
            oA, oB = carry
            j = g // (CH // 16)
            kk = lax.rem(g, CH // 16)
            vs = si_v[j, pl.ds(kk * 16, 16)]
            vA = dA_v[j, pl.ds(kk * 16, 16)]
            vB = dB_v[j, pl.ds(kk * 16, 16)]

            def half_step(vd, o, t_ref, s_ref):
                m = vd < half_n
                cs = jnp.where(m, 1, 0)
                for step in (1, 2, 4, 8):
                    idx = jnp.maximum(lanes - step, 0)
                    cs = cs + jnp.where(lanes >= step, jnp.take(cs, idx), 0)
                sel = jnp.zeros((16,), jnp.int32)
                for i in range(16):
                    sel = sel + jnp.where(cs[i] <= lanes, 1, 0)
                sel = jnp.minimum(sel, 15)
                t_ref[pl.ds(o, 16)] = jnp.take(vd, sel)
                s_ref[pl.ds(o, 16)] = jnp.take(vs, sel)
                return o + cs[15]

            oA = half_step(vA, oA, tA_v, sA_v)
            oB = half_step(vB, oB, tB_v, sB_v)
            return (oA, oB)

        oA, oB = lax.fori_loop(
            0, cpt * (CH // 16), grp, (jnp.int32(0), jnp.int32(0))
        )

        def pads(g, carry):
            oA_, oB_ = carry
            dv = half_n + lax.rem(g * 16 + lanes, dumpreg)
            z16 = jnp.zeros((16,), jnp.int32)
            tA_v[pl.ds(oA_, 16)] = dv
            sA_v[pl.ds(oA_, 16)] = z16
            tB_v[pl.ds(oB_, 16)] = dv
            sB_v[pl.ds(oB_, 16)] = z16
            return (oA_ + 16, oB_ + 16)

        lax.fori_loop(0, (4 * CH) // 16 + 1, pads, (oA, oB))

        def wc(kk, _):
            cnt_v[0, pl.ds(kk * 16, 16)] = jnp.full((16,), oA, jnp.int32)
            cnt_v[1, pl.ds(kk * 16, 16)] = jnp.full((16,), oB, jnp.int32)
            return 0

        lax.fori_loop(0, CH // 16, wc, 0)
        pltpu.sync_copy(sA_v.at[pl.ds(0, ept)], srcL_hbm.at[0, w])
        pltpu.sync_copy(tA_v.at[pl.ds(0, ept)], dstL_hbm.at[0, w])
        pltpu.sync_copy(sB_v.at[pl.ds(0, ept)], srcL_hbm.at[1, w])
        pltpu.sync_copy(tB_v.at[pl.ds(0, ept)], dstL_hbm.at[1, w])
        pltpu.sync_copy(cnt_v, cnt_hbm.at[w])

    srcL, dstL, cnts = k(srcp, dstA, dstB)
    return (srcL.reshape(2, NW, cpt, CH), dstL.reshape(2, NW, cpt, CH), cnts)


def _sc_aggregate(ts, srcL, dstL, cnts, acc_n, half_n, half_acc, cpt):
    d = ts.shape[1]

    @functools.partial(
        pl.kernel,
        out_type=jax.ShapeDtypeStruct((NC, acc_n, d), jnp.float32),
        mesh=_mesh(),
        scratch_types=[
            pltpu.VMEM((cpt, CH), jnp.int32),
            pltpu.VMEM((cpt, CH), jnp.int32),
            pltpu.VMEM((2, CH, d), jnp.float32),
            pltpu.VMEM((CH, d), jnp.float32),
            pltpu.VMEM((2, CH), jnp.int32),
            pltpu.VMEM_SHARED((half_acc, d), jnp.float32),
            pltpu.SemaphoreType.DMA((2,)),
        ],
    )
    def k(ts_hbm, srcL_hbm, dstL_hbm, cnt_hbm, out_hbm,
          si_v, di_v, rows_v, z_v, cnt_v, acc_sh, gsem):
        c = lax.axis_index("c")
        s = lax.axis_index("s")
        w = s * NC + c
        zrows_per_tile = half_acc // NS
        orows_per_tile = half_n // NS

        def zrow(r, _):
            def zcol(kk, _):
                z_v[r, pl.ds(kk * 16, 16)] = jnp.zeros((16,), jnp.float32)
                return 0

            lax.fori_loop(0, d // 16, zcol, 0)
            return 0

        lax.fori_loop(0, CH, zrow, 0)
        pltpu.sync_copy(cnt_hbm.at[w], cnt_v)

        for half in (0, 1):
            def zblk(i, _):
                pltpu.sync_copy(z_v, acc_sh.at[pl.ds(s * zrows_per_tile + i * CH, CH)])
                return 0

            lax.fori_loop(0, zrows_per_tile // CH, zblk, 0)
            pltpu.sync_copy(srcL_hbm.at[half, w], si_v)
            pltpu.sync_copy(dstL_hbm.at[half, w], di_v)
            plsc.subcore_barrier()

            cnt = cnt_v[half, pl.ds(0, 16)][0]
            npairs = jnp.maximum((cnt + 2 * CH - 1) // (2 * CH), 1)
            nch = npairs * 2

            pltpu.make_async_copy(
                ts_hbm.at[si_v.at[0]], rows_v.at[0], gsem.at[0]
            ).start()

            def body(i, _):
                j0 = i * 2
                pltpu.make_async_copy(
                    ts_hbm.at[si_v.at[j0 + 1]], rows_v.at[1], gsem.at[1]
                ).start()
                pltpu.make_async_copy(
                    ts_hbm.at[si_v.at[j0]], rows_v.at[0], gsem.at[0]
                ).wait()
                pltpu.sync_copy(rows_v.at[0], acc_sh.at[di_v.at[j0]], add=True)

                @pl.when(j0 + 2 < nch)
                def _():
                    pltpu.make_async_copy(
                        ts_hbm.at[si_v.at[j0 + 2]], rows_v.at[0], gsem.at[0]
                    ).start()

                pltpu.make_async_copy(
                    ts_hbm.at[si_v.at[j0 + 1]], rows_v.at[1], gsem.at[1]
                ).wait()
                pltpu.sync_copy(rows_v.at[1], acc_sh.at[di_v.at[j0 + 1]], add=True)
                return 0

            lax.fori_loop(0, npairs, body, 0)
            plsc.subcore_barrier()
            pltpu.sync_copy(
                acc_sh.at[pl.ds(s * orows_per_tile, orows_per_tile)],
                out_hbm.at[c, pl.ds(half * half_n + s * orows_per_tile,
                                    orows_per_tile)],
            )
            plsc.subcore_barrier()

    return k(ts, srcL, dstL, cnts)


def _tc_prep(deg2, dstp, n, half_n, half_acc):
    acc_n = deg2.shape[1]
    nw, cpt, ch = dstp.shape
    flat = (nw * cpt, ch)
    dstf = dstp.reshape(flat)

    def body(deg_ref, dst_ref, dinv_ref, dstA_ref, dstB_ref):
        dinv_ref[...] = lax.rsqrt(deg_ref[0:1, :] + deg_ref[1:2, :] + 1.0)
        dst = dst_ref[...]
        pos = (jax.lax.broadcasted_iota(jnp.int32, flat, 0) * ch
               + jax.lax.broadcasted_iota(jnp.int32, flat, 1))
        dump = half_n + lax.rem(pos, half_acc - half_n)
        dstA_ref[...] = jnp.where(dst < half_n, dst, dump)
        inB = jnp.logical_and(dst >= half_n, dst < n)
        dstB_ref[...] = jnp.where(inB, dst - half_n, dump)

    dinv, dstA, dstB = pl.pallas_call(
        body,
        out_shape=[
            jax.ShapeDtypeStruct((1, acc_n), jnp.float32),
            jax.ShapeDtypeStruct(flat, jnp.int32),
            jax.ShapeDtypeStruct(flat, jnp.int32),
        ],
    )(deg2, dstf)
    return dinv, dstA.reshape(nw, cpt, ch), dstB.reshape(nw, cpt, ch)


def _tc_first(x, W1, dinv, blk):
    n, d = x.shape

    def body(x_ref, w_ref, dv_ref, out_ref):
        out_ref[...] = (
            jnp.dot(x_ref[...], w_ref[...], preferred_element_type=jnp.float32)
            * dv_ref[...]
        )

    return pl.pallas_call(
        body,
        grid=(n // blk,),
        in_specs=[
            pl.BlockSpec((blk, d), lambda i: (i, 0)),
            pl.BlockSpec((d, d), lambda i: (0, 0)),
            pl.BlockSpec((blk, 1), lambda i: (i, 0)),
        ],
        out_specs=pl.BlockSpec((blk, d), lambda i: (i, 0)),
        out_shape=jax.ShapeDtypeStruct((n, d), jnp.float32),
    )(x, W1, dinv)


def _tc_layer(p, ts, dinv, b, W, blk):
    n, d = ts.shape

    def body(p_ref, ts_ref, dv_ref, b_ref, w_ref, out_ref):
        h = (p_ref[0] + p_ref[1] + ts_ref[...]) * dv_ref[...] + b_ref[...]
        h = jnp.maximum(h, 0.0)
        out_ref[...] = (
            jnp.dot(h, w_ref[...], preferred_element_type=jnp.float32) * dv_ref[...]
        )

    return pl.pallas_call(
        body,
        grid=(n // blk,),
        in_specs=[
            pl.BlockSpec((2, blk, d), lambda i: (0, i, 0)),
            pl.BlockSpec((blk, d), lambda i: (i, 0)),
            pl.BlockSpec((blk, 1), lambda i: (i, 0)),
            pl.BlockSpec((1, d), lambda i: (0, 0)),
            pl.BlockSpec((d, d), lambda i: (0, 0)),
        ],
        out_specs=pl.BlockSpec((blk, d), lambda i: (i, 0)),
        out_shape=jax.ShapeDtypeStruct((n, d), jnp.float32),
    )(p, ts, dinv, b, W)


def _tc_head(p, ts, dinv, b3, L1w, L1b, L2w, L2b, blk):
    n, d = ts.shape
    g_steps = n // blk

    def body(p_ref, ts_ref, dv_ref, b_ref, l1w_ref, l1b_ref, l2w_ref, l2b_ref,
             out_ref, acc_ref):
        i = pl.program_id(0)
        h = (p_ref[0] + p_ref[1] + ts_ref[...]) * dv_ref[...] + b_ref[...]
        h = jnp.maximum(h, 0.0)
        bsum = jnp.sum(h, axis=0, keepdims=True)

        @pl.when(i == 0)
        def _():
            acc_ref[...] = bsum

        @pl.when(i > 0)
        def _():
            acc_ref[...] = acc_ref[...] + bsum

        @pl.when(i == g_steps - 1)
        def _():
            g = acc_ref[...] * (1.0 / n)
            z = jnp.dot(g, l1w_ref[...], preferred_element_type=jnp.float32)
            z = jnp.maximum(z + l1b_ref[...], 0.0)
            o = jnp.dot(z, l2w_ref[...], preferred_element_type=jnp.float32)
            out_ref[...] = jax.nn.sigmoid(o + l2b_ref[...])

    return pl.pallas_call(
        body,
        grid=(g_steps,),
        in_specs=[
            pl.BlockSpec((2, blk, d), lambda i: (0, i, 0)),
            pl.BlockSpec((blk, d), lambda i: (i, 0)),
            pl.BlockSpec((blk, 1), lambda i: (i, 0)),
            pl.BlockSpec((1, d), lambda i: (0, 0)),
            pl.BlockSpec((d, d), lambda i: (0, 0)),
            pl.BlockSpec((1, d), lambda i: (0, 0)),
            pl.BlockSpec((d, 1), lambda i: (0, 0)),
            pl.BlockSpec((1, 1), lambda i: (0, 0)),
        ],
        out_specs=pl.BlockSpec((1, 1), lambda i: (0, 0)),
        out_shape=jax.ShapeDtypeStruct((1, 1), jnp.float32),
        scratch_shapes=[pltpu.VMEM((1, d), jnp.float32)],
    )(p, ts, dinv, b3, L1w, L1b, L2w, L2b)


def kernel(x, edge_index, W1, b1, W2, b2, W3, b3, L1w, L1b, L2w, L2b):
    n, d = x.shape
    e = edge_index.shape[1]
    blk = 2000

    ept = ((e + NW * CH - 1) // (NW * CH)) * CH
    if (ept // CH) % 2:
        ept += CH
    cpt = ept // CH
    epad = NW * ept
    acc_n = ((n + NS * CH) // (NS * CH)) * NS * CH

    src = edge_index[0]
    dst = edge_index[1]
    pad = epad - e
    srcp = jnp.concatenate([src, jnp.zeros((pad,), jnp.int32)]).reshape(NW, cpt, CH)
    dstp = jnp.concatenate([dst, jnp.full((pad,), n, jnp.int32)]).reshape(NW, cpt, CH)

    half_n = acc_n // 2
    half_acc = ((half_n + 1 + NS * CH - 1) // (NS * CH)) * (NS * CH)

    deg2 = _sc_degree(dstp, n, acc_n, cpt)
    dinv, dstA, dstB = _tc_prep(deg2, dstp, n, half_n, half_acc)
    dinv = dinv.reshape(acc_n, 1)
    srcL, dstL, cnts = _sc_partition(srcp, dstA, dstB, half_n, half_acc, cpt)

    ts = _tc_first(x, W1, dinv, blk)
    p = _sc_aggregate(ts, srcL, dstL, cnts, acc_n, half_n, half_acc, cpt)
    ts = _tc_layer(p, ts, dinv, b1.reshape(1, d), W2, blk)
    p = _sc_aggregate(ts, srcL, dstL, cnts, acc_n, half_n, half_acc, cpt)
    ts = _tc_layer(p, ts, dinv, b2.reshape(1, d), W3, blk)
    p = _sc_aggregate(ts, srcL, dstL, cnts, acc_n, half_n, half_acc, cpt)
    out = _tc_head(p, ts, dinv, b3.reshape(1, d), L1w, L1b.reshape(1, d),
                   L2w, L2b.reshape(1, 1), blk)
    return out.reshape(1)


# speedup vs baseline: 2.0881x; 1.0593x over previous
import functools

import jax
import jax.numpy as jnp
from jax import lax
from jax.experimental import pallas as pl
from jax.experimental.pallas import tpu as pltpu
from jax.experimental.pallas import tpu_sc as plsc

NC = 2
NS = 16
NW = NC * NS
CH = 128


def _mesh():
    return plsc.VectorSubcoreMesh(core_axis_name="c", subcore_axis_name="s")


def _sc_degree(dstp, n, acc_n, cpt):
    del n
    @functools.partial(
        pl.kernel,
        out_type=jax.ShapeDtypeStruct((NC, acc_n), jnp.float32),
        mesh=_mesh(),
        scratch_types=[
            pltpu.VMEM((cpt, CH), jnp.int32),
            pltpu.VMEM((CH,), jnp.float32),
            pltpu.VMEM((acc_n // NS,), jnp.float32),
            pltpu.VMEM_SHARED((acc_n,), jnp.float32),
        ],
    )
    def k(dst_hbm, out_hbm, idx_v, ones_v, z_v, deg_sh):
        c = lax.axis_index("c")
        s = lax.axis_index("s")
        w = s * NC + c
        zslice = acc_n // NS

        def fill_ones(i, _):
            ones_v[pl.ds(i * 16, 16)] = jnp.ones((16,), jnp.float32)
            return 0

        lax.fori_loop(0, CH // 16, fill_ones, 0)

        def fill_zeros(i, _):
            z_v[pl.ds(i * 16, 16)] = jnp.zeros((16,), jnp.float32)
            return 0

        lax.fori_loop(0, zslice // 16, fill_zeros, 0)

        pltpu.sync_copy(z_v, deg_sh.at[pl.ds(s * zslice, zslice)])
        pltpu.sync_copy(dst_hbm.at[w], idx_v)
        plsc.subcore_barrier()

        def body(j, _):
            pltpu.sync_copy(ones_v, deg_sh.at[idx_v.at[j]], add=True)
            return 0

        lax.fori_loop(0, cpt, body, 0)
        plsc.subcore_barrier()

        @pl.when(s == 0)
        def _():
            pltpu.sync_copy(deg_sh.at[pl.ds(0, acc_n)], out_hbm.at[c])

    return k(dstp)


def _tc_positions(dst3, bk, nbk, sub):
    nblk, cs, _ = dst3.shape
    steps = nblk // sub

    def mk_consts():
        row_i = jax.lax.broadcasted_iota(jnp.int32, (cs, cs), 0)
        col_i = jax.lax.broadcasted_iota(jnp.int32, (cs, cs), 1)
        tri = jnp.where(col_i < row_i, 1.0, 0.0)
        kio = jax.lax.broadcasted_iota(jnp.int32, (1, nbk), 1)
        return tri, kio

    def body1(dst_ref, pos_ref, pbc_ref, run_ref):
        i = pl.program_id(0)
        tri, kio = mk_consts()

        @pl.when(i == 0)
        def _():
            run_ref[...] = jnp.zeros((1, nbk), jnp.float32)

        run = run_ref[...]
        for j in range(sub):
            oh = jnp.where(dst_ref[j] // bk == kio, 1.0, 0.0)
            tcum = jnp.dot(tri, oh, preferred_element_type=jnp.float32)
            posl = jnp.sum(oh * (tcum + run), axis=1, keepdims=True)
            pos_ref[j] = posl.astype(jnp.int32)
            run = run + jnp.sum(oh, axis=0, keepdims=True)
        run_ref[...] = run

        @pl.when(i == steps - 1)
        def _():
            caps = jnp.floor((run + 127.0) * (1.0 / 128.0)) * 128.0
            bio_r = jax.lax.broadcasted_iota(jnp.int32, (nbk, nbk), 0)
            bio_c = jax.lax.broadcasted_iota(jnp.int32, (nbk, nbk), 1)
            upper = jnp.where(bio_r < bio_c, 1.0, 0.0)
            ident = jnp.where(bio_r == bio_c, 1.0, 0.0)
            pbrow = jnp.dot(caps, upper, preferred_element_type=jnp.float32)
            col_dn = (((1,), (1,)), ((), ()))
            pbcol = lax.dot_general(ident, pbrow, col_dn,
                                    preferred_element_type=jnp.float32)
            cntcol = lax.dot_general(ident, run, col_dn,
                                     preferred_element_type=jnp.float32)
            pbc_ref[0:nbk, :] = jnp.broadcast_to(
                pbcol, (nbk, 128)).astype(jnp.int32)
            pbc_ref[nbk:2 * nbk, :] = jnp.broadcast_to(
                cntcol, (nbk, 128)).astype(jnp.int32)
            pbc_ref[2 * nbk:8, :] = jnp.zeros((8 - 2 * nbk, 128), jnp.int32)

    posl3, pbc = pl.pallas_call(
        body1,
        grid=(steps,),
        in_specs=[pl.BlockSpec((sub, cs, 1), lambda i: (i, 0, 0))],
        out_specs=[pl.BlockSpec((sub, cs, 1), lambda i: (i, 0, 0)),
                   pl.BlockSpec((8, 128), lambda i: (0, 0))],
        out_shape=[
            jax.ShapeDtypeStruct((nblk, cs, 1), jnp.int32),
            jax.ShapeDtypeStruct((8, 128), jnp.int32),
        ],
        scratch_shapes=[pltpu.VMEM((1, nbk), jnp.float32)],
    )(dst3)

    def body2(posl_ref, dst_ref, pbc_ref, pos_ref):
        _, kio = mk_consts()
        pbcol = pbc_ref[0:nbk, 0:1].astype(jnp.float32)
        for j in range(sub):
            oh = jnp.where(dst_ref[j] // bk == kio, 1.0, 0.0)
            pbsel = jnp.dot(oh, pbcol, preferred_element_type=jnp.float32)
            pos_ref[j] = (posl_ref[j].astype(jnp.float32) + pbsel).astype(jnp.int32)

    pos3 = pl.pallas_call(
        body2,
        grid=(steps,),
        in_specs=[pl.BlockSpec((sub, cs, 1), lambda i: (i, 0, 0)),
                  pl.BlockSpec((sub, cs, 1), lambda i: (i, 0, 0)),
                  pl.BlockSpec((8, 128), lambda i: (0, 0))],
        out_specs=pl.BlockSpec((sub, cs, 1), lambda i: (i, 0, 0)),
        out_shape=jax.ShapeDtypeStruct((nblk, cs, 1), jnp.int32),
    )(posl3, dst3, pbc)
    return pos3, pbc


def _sc_reorder(srcp2, dstp2, posp, sort_rd, cpt):
    @functools.partial(
        pl.kernel,
        out_type=(
            jax.ShapeDtypeStruct((NC, sort_rd), jnp.int32),
            jax.ShapeDtypeStruct((NC, sort_rd), jnp.int32),
        ),
        mesh=_mesh(),
        scratch_types=[
            pltpu.VMEM((cpt, CH), jnp.int32),
            pltpu.VMEM((cpt, CH), jnp.int32),
            pltpu.VMEM((cpt, CH), jnp.int32),
            pltpu.VMEM((2048,), jnp.int32),
            pltpu.VMEM_SHARED((sort_rd,), jnp.int32),
            pltpu.VMEM_SHARED((sort_rd,), jnp.int32),
        ],
    )
    def k(src_hbm, dst_hbm, pos_hbm, outS_hbm, outD_hbm,
          si_v, di_v, po_v, z_v, srcS_sh, dstS_sh):
        c = lax.axis_index("c")
        s = lax.axis_index("s")
        w = s * NC + c
        zpt = sort_rd // NS

        def zf(i, _):
            z_v[pl.ds(i * 16, 16)] = jnp.zeros((16,), jnp.int32)
            return 0

        lax.fori_loop(0, 2048 // 16, zf, 0)

        def zs(i, _):
            pltpu.sync_copy(z_v, srcS_sh.at[pl.ds(s * zpt + i * 2048, 2048)])
            pltpu.sync_copy(z_v, dstS_sh.at[pl.ds(s * zpt + i * 2048, 2048)])
            return 0

        lax.fori_loop(0, zpt // 2048, zs, 0)
        pltpu.sync_copy(src_hbm.at[w], si_v)
        pltpu.sync_copy(dst_hbm.at[w], di_v)
        pltpu.sync_copy(pos_hbm.at[w], po_v)
        plsc.subcore_barrier()

        def body(j, _):
            pltpu.sync_copy(si_v.at[j], srcS_sh.at[po_v.at[j]], add=True)
            pltpu.sync_copy(di_v.at[j], dstS_sh.at[po_v.at[j]], add=True)
            return 0

        lax.fori_loop(0, cpt, body, 0)
        plsc.subcore_barrier()
        pltpu.sync_copy(srcS_sh.at[pl.ds(s * zpt, zpt)],
                        outS_hbm.at[c, pl.ds(s * zpt, zpt)])
        pltpu.sync_copy(dstS_sh.at[pl.ds(s * zpt, zpt)],
                        outD_hbm.at[c, pl.ds(s * zpt, zpt)])

    return k(srcp2, dstp2, posp)


def _tc_merge(pS, pD, pbc, half_n, dumpn, blkr):
    nc, rows, lanes = pS.shape
    steps = rows // blkr

    def body(s_ref, d_ref, pbc_ref, os_ref, od_ref):
        ib = pl.program_id(0)
        base1 = pbc_ref[1, 0]
        cnt0 = pbc_ref[2, 0]
        cnt1 = pbc_ref[3, 0]
        sm = jnp.maximum(s_ref[0], s_ref[1])
        dm = jnp.maximum(d_ref[0], d_ref[1])
        ri = jax.lax.broadcasted_iota(jnp.int32, (blkr, lanes), 0)
        li = jax.lax.broadcasted_iota(jnp.int32, (blkr, lanes), 1)
        gi = (ib * blkr + ri) * lanes + li
        reg1 = gi >= base1
        local = gi - jnp.where(reg1, base1, 0)
        dead = local >= jnp.where(reg1, cnt1, cnt0)
        dmr = dm - jnp.where(reg1, half_n, 0)
        od_ref[...] = jnp.where(dead, half_n + jnp.remainder(gi, dumpn), dmr)
        os_ref[...] = jnp.where(dead, 0, sm)

    del nc
    return pl.pallas_call(
        body,
        grid=(steps,),
        in_specs=[
            pl.BlockSpec((2, blkr, lanes), lambda i: (0, i, 0)),
            pl.BlockSpec((2, blkr, lanes), lambda i: (0, i, 0)),
            pl.BlockSpec((8, 128), lambda i: (0, 0)),
        ],
        out_specs=[pl.BlockSpec((blkr, lanes), lambda i: (i, 0)),
                   pl.BlockSpec((blkr, lanes), lambda i: (i, 0))],
        out_shape=[
            jax.ShapeDtypeStruct((rows, lanes), jnp.int32),
            jax.ShapeDtypeStruct((rows, lanes), jnp.int32),
        ],
    )(pS, pD, pbc)


def _sc_aggregate(ts, srcS, dstS, pbc, acc_n, half_n, half_acc):
    d = ts.shape[1]

    @functools.partial(
        pl.kernel,
        out_type=jax.ShapeDtypeStruct((acc_n, d), jnp.float32),
        mesh=_mesh(),
        scratch_types=[
            pltpu.VMEM((2, CH, d), jnp.float32),
            pltpu.VMEM((CH, d), jnp.float32),
            pltpu.VMEM((2, CH), jnp.int32),
            pltpu.VMEM((2, CH), jnp.int32),
            pltpu.VMEM((8, CH), jnp.int32),
            pltpu.VMEM_SHARED((half_acc, d), jnp.float32),
            pltpu.SemaphoreType.DMA((2,)),
        ],
    )
    def k(ts_hbm, srcS_hbm, dstS_hbm, pbc_hbm, out_hbm,
          rows_v, z_v, si_v, di_v, pb_v, acc_sh, gsem):
        c = lax.axis_index("c")
        s = lax.axis_index("s")
        zrows = half_acc // NS
        orows = half_n // NS

        def zrow(r, _):
            def zcol(kk, _):
                z_v[r, pl.ds(kk * 16, 16)] = jnp.zeros((16,), jnp.float32)
                return 0

            lax.fori_loop(0, d // 16, zcol, 0)
            return 0

        lax.fori_loop(0, CH, zrow, 0)

        def zblk(i, _):
            pltpu.sync_copy(z_v, acc_sh.at[pl.ds(s * zrows + i * CH, CH)])
            return 0

        lax.fori_loop(0, zrows // CH, zblk, 0)

        pltpu.sync_copy(pbc_hbm, pb_v)
        base1 = pb_v[1, pl.ds(0, 16)][0]
        cnt0 = pb_v[2, pl.ds(0, 16)][0]
        cnt1 = pb_v[3, pl.ds(0, 16)][0]
        base_c = jnp.where(c == 0, 0, base1)
        cnt_c = jnp.where(c == 0, cnt0, cnt1)
        nch = (cnt_c + CH - 1) // CH
        q = nch // NS
        rem = nch - q * NS
        myn = q + jnp.where(s < rem, 1, 0)
        g0q = base_c // CH + s * q + jnp.minimum(s, rem)
        plsc.subcore_barrier()

        @pl.when(myn > 0)
        def _():
            pltpu.sync_copy(srcS_hbm.at[pl.ds(g0q * CH, CH)], si_v.at[0])
            pltpu.sync_copy(dstS_hbm.at[pl.ds(g0q * CH, CH)], di_v.at[0])
            pltpu.make_async_copy(
                ts_hbm.at[si_v.at[0]], rows_v.at[0], gsem.at[0]
            ).start()

            def body(i, _):
                p = lax.rem(i, 2)

                @pl.when(i + 1 < myn)
                def _():
                    pn = lax.rem(i + 1, 2)
                    g1 = (g0q + i + 1) * CH
                    pltpu.sync_copy(srcS_hbm.at[pl.ds(g1, CH)], si_v.at[pn])
                    pltpu.sync_copy(dstS_hbm.at[pl.ds(g1, CH)], di_v.at[pn])
                    pltpu.make_async_copy(
                        ts_hbm.at[si_v.at[pn]], rows_v.at[pn], gsem.at[pn]
                    ).start()

                pltpu.make_async_copy(
                    ts_hbm.at[si_v.at[p]], rows_v.at[p], gsem.at[p]
                ).wait()
                pltpu.sync_copy(rows_v.at[p], acc_sh.at[di_v.at[p]], add=True)
                return 0

            lax.fori_loop(0, myn, body, 0)

        plsc.subcore_barrier()
        pltpu.sync_copy(
            acc_sh.at[pl.ds(s * orows, orows)],
            out_hbm.at[pl.ds((c * (half_n // 8) + s * (orows // 8)) * 8,
                             orows)],
        )

    return k(ts, srcS, dstS, pbc)


def _tc_prep(deg2):
    acc_n = deg2.shape[1]

    def body(deg_ref, dinv_ref):
        dinv_ref[...] = lax.rsqrt(deg_ref[0:1, :] + deg_ref[1:2, :] + 1.0)

    return pl.pallas_call(
        body, out_shape=jax.ShapeDtypeStruct((1, acc_n), jnp.float32)
    )(deg2)


def _tc_first(x, W1, dinv, blk):
    n, d = x.shape

    def body(x_ref, w_ref, dv_ref, out_ref):
        out_ref[...] = (
            jnp.dot(x_ref[...], w_ref[...], preferred_element_type=jnp.float32)
            * dv_ref[...]
        )

    return pl.pallas_call(
        body,
        grid=(n // blk,),
        in_specs=[
            pl.BlockSpec((blk, d), lambda i: (i, 0)),
            pl.BlockSpec((d, d), lambda i: (0, 0)),
            pl.BlockSpec((blk, 1), lambda i: (i, 0)),
        ],
        out_specs=pl.BlockSpec((blk, d), lambda i: (i, 0)),
        out_shape=jax.ShapeDtypeStruct((n, d), jnp.float32),
    )(x, W1, dinv)


def _tc_layer(p, ts, dinv, b, W, blk):
    n, d = ts.shape

    def body(p_ref, ts_ref, dv_ref, b_ref, w_ref, out_ref):
        h = (p_ref[...] + ts_ref[...]) * dv_ref[...] + b_ref[...]
        h = jnp.maximum(h, 0.0)
        out_ref[...] = (
            jnp.dot(h, w_ref[...], preferred_element_type=jnp.float32) * dv_ref[...]
        )

    return pl.pallas_call(
        body,
        grid=(n // blk,),
        in_specs=[
            pl.BlockSpec((blk, d), lambda i: (i, 0)),
            pl.BlockSpec((blk, d), lambda i: (i, 0)),
            pl.BlockSpec((blk, 1), lambda i: (i, 0)),
            pl.BlockSpec((1, d), lambda i: (0, 0)),
            pl.BlockSpec((d, d), lambda i: (0, 0)),
        ],
        out_specs=pl.BlockSpec((blk, d), lambda i: (i, 0)),
        out_shape=jax.ShapeDtypeStruct((n, d), jnp.float32),
    )(p, ts, dinv, b, W)


def _tc_head(p, ts, dinv, b3, L1w, L1b, L2w, L2b, blk, n_real):
    n, d = ts.shape
    g_steps = n // blk

    def body(p_ref, ts_ref, dv_ref, b_ref, l1w_ref, l1b_ref, l2w_ref, l2b_ref,
             out_ref, acc_ref):
        i = pl.program_id(0)
        h = (p_ref[...] + ts_ref[...]) * dv_ref[...] + b_ref[...]
        h = jnp.maximum(h, 0.0)
        bsum = jnp.sum(h, axis=0, keepdims=True)

        @pl.when(i == 0)
        def _():
            acc_ref[...] = bsum

        @pl.when(i > 0)
        def _():
            acc_ref[...] = acc_ref[...] + bsum

        @pl.when(i == g_steps - 1)
        def _():
            g = acc_ref[...] * (1.0 / n_real)
            z = jnp.dot(g, l1w_ref[...], preferred_element_type=jnp.float32)
            z = jnp.maximum(z + l1b_ref[...], 0.0)
            o = jnp.dot(z, l2w_ref[...], preferred_element_type=jnp.float32)
            out_ref[...] = jax.nn.sigmoid(o + l2b_ref[...])

    return pl.pallas_call(
        body,
        grid=(g_steps,),
        in_specs=[
            pl.BlockSpec((blk, d), lambda i: (i, 0)),
            pl.BlockSpec((blk, d), lambda i: (i, 0)),
            pl.BlockSpec((blk, 1), lambda i: (i, 0)),
            pl.BlockSpec((1, d), lambda i: (0, 0)),
            pl.BlockSpec((d, d), lambda i: (0, 0)),
            pl.BlockSpec((1, d), lambda i: (0, 0)),
            pl.BlockSpec((d, 1), lambda i: (0, 0)),
            pl.BlockSpec((1, 1), lambda i: (0, 0)),
        ],
        out_specs=pl.BlockSpec((1, 1), lambda i: (0, 0)),
        out_shape=jax.ShapeDtypeStruct((1, 1), jnp.float32),
        scratch_shapes=[pltpu.VMEM((1, d), jnp.float32)],
    )(p, ts, dinv, b3, L1w, L1b, L2w, L2b)


def kernel(x, edge_index, W1, b1, W2, b2, W3, b3, L1w, L1b, L2w, L2b):
    n, d = x.shape
    e = edge_index.shape[1]
    blk = 2000

    ept = ((e + NW * CH - 1) // (NW * CH)) * CH
    cpt = ept // CH
    epad = NW * ept
    acc_n = ((n + NS * CH) // (NS * CH)) * NS * CH
    half_n = acc_n // 2
    dumpn = 1024
    half_acc = half_n + dumpn
    nbk = 2
    cs = 256
    nblk = e // cs
    sort_sz = e + NW * CH
    sort_rd = ((sort_sz + 256 + NS * 2048 - 1) // (NS * 2048)) * (NS * 2048)

    src = edge_index[0]
    dst = edge_index[1]
    pad = epad - e
    dstp = jnp.concatenate([dst, jnp.full((pad,), n, jnp.int32)]).reshape(
        NW, cpt, CH)

    deg2 = _sc_degree(dstp, n, acc_n, cpt)
    dinv = _tc_prep(deg2).reshape(acc_n, 1)

    dst3 = dst.reshape(nblk, cs, 1)
    pos3, pbc = _tc_positions(dst3, half_n, nbk, 10)
    posf = pos3.reshape(e)
    dump0 = sort_sz
    padpos = dump0 + jnp.arange(pad, dtype=jnp.int32)
    srcp2 = jnp.concatenate([src, jnp.zeros((pad,), jnp.int32)]).reshape(
        NW, cpt, CH)
    dstp2 = jnp.concatenate([dst, jnp.zeros((pad,), jnp.int32)]).reshape(
        NW, cpt, CH)
    posp = jnp.concatenate([posf, padpos]).reshape(NW, cpt, CH)
    pS, pD = _sc_reorder(srcp2, dstp2, posp, sort_rd, cpt)
    srcS2, dstS2 = _tc_merge(pS.reshape(NC, sort_rd // 128, 128),
                             pD.reshape(NC, sort_rd // 128, 128),
                             pbc, half_n, dumpn, 256)
    srcS = srcS2.reshape(sort_rd)
    dstS = dstS2.reshape(sort_rd)

    ts = _tc_first(x, W1, dinv, blk)
    p = _sc_aggregate(ts, srcS, dstS, pbc, acc_n, half_n, half_acc)
    ts = _tc_layer(p, ts, dinv, b1.reshape(1, d), W2, blk)
    p = _sc_aggregate(ts, srcS, dstS, pbc, acc_n, half_n, half_acc)
    ts = _tc_layer(p, ts, dinv, b2.reshape(1, d), W3, blk)
    p = _sc_aggregate(ts, srcS, dstS, pbc, acc_n, half_n, half_acc)
    out = _tc_head(p, ts, dinv, b3.reshape(1, d), L1w, L1b.reshape(1, d),
                   L2w, L2b.reshape(1, 1), blk, n)
    return out.reshape(1)


# speedup vs baseline: 2.4872x; 1.1911x over previous
import functools

import jax
import jax.numpy as jnp
from jax import lax
from jax.experimental import pallas as pl
from jax.experimental.pallas import tpu as pltpu
from jax.experimental.pallas import tpu_sc as plsc

NC = 2
NS = 16
NW = NC * NS
CH = 128


def _mesh():
    return plsc.VectorSubcoreMesh(core_axis_name="c", subcore_axis_name="s")


def _sc_sortdeg(srcp, dstp, posp, pbc, n, acc_n, half_n, sort_rd, cpt):
    @functools.partial(
        pl.kernel,
        out_type=(
            jax.ShapeDtypeStruct((NC, acc_n), jnp.float32),
            jax.ShapeDtypeStruct((NC, sort_rd), jnp.int32),
            jax.ShapeDtypeStruct((NC, sort_rd), jnp.int32),
        ),
        mesh=_mesh(),
        scratch_types=[
            pltpu.VMEM((cpt, CH), jnp.int32),
            pltpu.VMEM((cpt, CH), jnp.int32),
            pltpu.VMEM((cpt, CH), jnp.int32),
            pltpu.VMEM((CH,), jnp.float32),
            pltpu.VMEM((2048,), jnp.int32),
            pltpu.VMEM((acc_n // NS,), jnp.float32),
            pltpu.VMEM((8, CH), jnp.int32),
            pltpu.VMEM_SHARED((acc_n,), jnp.float32),
            pltpu.VMEM_SHARED((sort_rd,), jnp.int32),
            pltpu.VMEM_SHARED((sort_rd,), jnp.int32),
        ],
    )
    def k(src_hbm, dst_hbm, pos_hbm, pbc_hbm, deg_hbm, outS_hbm, outD_hbm,
          si_v, di_v, po_v, ones_v, z_v, zf_v, pb_v, deg_sh, srcS_sh, dstS_sh):
        c = lax.axis_index("c")
        s = lax.axis_index("s")
        w = s * NC + c
        zpt = sort_rd // NS
        dslice = acc_n // NS

        def fo(i, _):
            ones_v[pl.ds(i * 16, 16)] = jnp.ones((16,), jnp.float32)
            return 0

        lax.fori_loop(0, CH // 16, fo, 0)

        def zf(i, _):
            z_v[pl.ds(i * 16, 16)] = jnp.zeros((16,), jnp.int32)
            return 0

        lax.fori_loop(0, 2048 // 16, zf, 0)

        def zff(i, _):
            zf_v[pl.ds(i * 16, 16)] = jnp.zeros((16,), jnp.float32)
            return 0

        lax.fori_loop(0, dslice // 16, zff, 0)

        def zs(i, _):
            pltpu.sync_copy(z_v, srcS_sh.at[pl.ds(s * zpt + i * 2048, 2048)])
            pltpu.sync_copy(z_v, dstS_sh.at[pl.ds(s * zpt + i * 2048, 2048)])
            return 0

        lax.fori_loop(0, zpt // 2048, zs, 0)
        pltpu.sync_copy(zf_v, deg_sh.at[pl.ds(s * dslice, dslice)])
        pltpu.sync_copy(src_hbm.at[w], si_v)
        pltpu.sync_copy(dst_hbm.at[w], di_v)
        pltpu.sync_copy(pos_hbm.at[w], po_v)
        pltpu.sync_copy(pbc_hbm, pb_v)
        base1 = pb_v[1, pl.ds(0, 16)][0]
        cnt1 = pb_v[3, pl.ds(0, 16)][0]

        def fix(g, _):
            j = g // (CH // 16)
            kk = lax.rem(g, CH // 16)
            vd = di_v[j, pl.ds(kk * 16, 16)]
            vp = po_v[j, pl.ds(kk * 16, 16)]
            add = (jnp.where(vd >= half_n, base1, 0)
                   + jnp.where(vd >= n, cnt1, 0))
            po_v[j, pl.ds(kk * 16, 16)] = vp + add
            return 0

        lax.fori_loop(0, cpt * (CH // 16), fix, 0)
        plsc.subcore_barrier()

        def body(j, _):
            pltpu.sync_copy(ones_v, deg_sh.at[di_v.at[j]], add=True)
            pltpu.sync_copy(si_v.at[j], srcS_sh.at[po_v.at[j]], add=True)
            pltpu.sync_copy(di_v.at[j], dstS_sh.at[po_v.at[j]], add=True)
            return 0

        lax.fori_loop(0, cpt, body, 0)
        plsc.subcore_barrier()
        pltpu.sync_copy(deg_sh.at[pl.ds(s * dslice, dslice)],
                        deg_hbm.at[c, pl.ds(s * dslice, dslice)])
        pltpu.sync_copy(srcS_sh.at[pl.ds(s * zpt, zpt)],
                        outS_hbm.at[c, pl.ds(s * zpt, zpt)])
        pltpu.sync_copy(dstS_sh.at[pl.ds(s * zpt, zpt)],
                        outD_hbm.at[c, pl.ds(s * zpt, zpt)])

    return k(srcp, dstp, posp, pbc)


def _tc_positions(dst3, bk, nbk, sub):
    nblk, cs, _ = dst3.shape
    steps = nblk // sub

    def mk_consts():
        row_i = jax.lax.broadcasted_iota(jnp.int32, (cs, cs), 0)
        col_i = jax.lax.broadcasted_iota(jnp.int32, (cs, cs), 1)
        tri = jnp.where(col_i < row_i, 1.0, 0.0)
        kio = jax.lax.broadcasted_iota(jnp.int32, (1, nbk), 1)
        return tri, kio

    def body1(dst_ref, pos_ref, pbc_ref, run_ref):
        i = pl.program_id(0)
        tri, kio = mk_consts()

        @pl.when(i == 0)
        def _():
            run_ref[...] = jnp.zeros((1, nbk), jnp.float32)

        run = run_ref[...]
        for j in range(sub):
            oh = jnp.where(dst_ref[j] // bk == kio, 1.0, 0.0)
            tcum = jnp.dot(tri, oh, preferred_element_type=jnp.float32)
            posl = jnp.sum(oh * (tcum + run), axis=1, keepdims=True)
            pos_ref[j] = posl.astype(jnp.int32)
            run = run + jnp.sum(oh, axis=0, keepdims=True)
        run_ref[...] = run

        @pl.when(i == steps - 1)
        def _():
            caps = jnp.floor((run + 127.0) * (1.0 / 128.0)) * 128.0
            bio_r = jax.lax.broadcasted_iota(jnp.int32, (nbk, nbk), 0)
            bio_c = jax.lax.broadcasted_iota(jnp.int32, (nbk, nbk), 1)
            upper = jnp.where(bio_r < bio_c, 1.0, 0.0)
            ident = jnp.where(bio_r == bio_c, 1.0, 0.0)
            pbrow = jnp.dot(caps, upper, preferred_element_type=jnp.float32)
            col_dn = (((1,), (1,)), ((), ()))
            pbcol = lax.dot_general(ident, pbrow, col_dn,
                                    preferred_element_type=jnp.float32)
            cntcol = lax.dot_general(ident, run, col_dn,
                                     preferred_element_type=jnp.float32)
            pbc_ref[0:nbk, :] = jnp.broadcast_to(
                pbcol, (nbk, 128)).astype(jnp.int32)
            pbc_ref[nbk:2 * nbk, :] = jnp.broadcast_to(
                cntcol, (nbk, 128)).astype(jnp.int32)
            pbc_ref[2 * nbk:8, :] = jnp.zeros((8 - 2 * nbk, 128), jnp.int32)

    posl3, pbc = pl.pallas_call(
        body1,
        grid=(steps,),
        in_specs=[pl.BlockSpec((sub, cs, 1), lambda i: (i, 0, 0))],
        out_specs=[pl.BlockSpec((sub, cs, 1), lambda i: (i, 0, 0)),
                   pl.BlockSpec((8, 128), lambda i: (0, 0))],
        out_shape=[
            jax.ShapeDtypeStruct((nblk, cs, 1), jnp.int32),
            jax.ShapeDtypeStruct((8, 128), jnp.int32),
        ],
        scratch_shapes=[pltpu.VMEM((1, nbk), jnp.float32)],
    )(dst3)

    return posl3, pbc


def _tc_merge(pS, pD, pbc, half_n, dumpn, blkr):
    nc, rows, lanes = pS.shape
    steps = rows // blkr

    def body(s_ref, d_ref, pbc_ref, os_ref, od_ref):
        ib = pl.program_id(0)
        base1 = pbc_ref[1, 0]
        cnt0 = pbc_ref[2, 0]
        cnt1 = pbc_ref[3, 0]
        sm = jnp.maximum(s_ref[0], s_ref[1])
        dm = jnp.maximum(d_ref[0], d_ref[1])
        ri = jax.lax.broadcasted_iota(jnp.int32, (blkr, lanes), 0)
        li = jax.lax.broadcasted_iota(jnp.int32, (blkr, lanes), 1)
        gi = (ib * blkr + ri) * lanes + li
        reg1 = gi >= base1
        local = gi - jnp.where(reg1, base1, 0)
        dead = local >= jnp.where(reg1, cnt1, cnt0)
        dmr = dm - jnp.where(reg1, half_n, 0)
        od_ref[...] = jnp.where(dead, half_n + jnp.remainder(gi, dumpn), dmr)
        os_ref[...] = jnp.where(dead, 0, sm)

    del nc
    return pl.pallas_call(
        body,
        grid=(steps,),
        in_specs=[
            pl.BlockSpec((2, blkr, lanes), lambda i: (0, i, 0)),
            pl.BlockSpec((2, blkr, lanes), lambda i: (0, i, 0)),
            pl.BlockSpec((8, 128), lambda i: (0, 0)),
        ],
        out_specs=[pl.BlockSpec((blkr, lanes), lambda i: (i, 0)),
                   pl.BlockSpec((blkr, lanes), lambda i: (i, 0))],
        out_shape=[
            jax.ShapeDtypeStruct((rows, lanes), jnp.int32),
            jax.ShapeDtypeStruct((rows, lanes), jnp.int32),
        ],
    )(pS, pD, pbc)


def _sc_aggregate(ts, srcS, dstS, pbc, acc_n, half_n, half_acc):
    d = ts.shape[1]

    @functools.partial(
        pl.kernel,
        out_type=jax.ShapeDtypeStruct((acc_n, d), jnp.float32),
        mesh=_mesh(),
        scratch_types=[
            pltpu.VMEM((2, CH, d), jnp.float32),
            pltpu.VMEM((CH, d), jnp.float32),
            pltpu.VMEM((2, CH), jnp.int32),
            pltpu.VMEM((2, CH), jnp.int32),
            pltpu.VMEM((8, CH), jnp.int32),
            pltpu.VMEM_SHARED((half_acc, d), jnp.float32),
            pltpu.SemaphoreType.DMA((2,)),
        ],
    )
    def k(ts_hbm, srcS_hbm, dstS_hbm, pbc_hbm, out_hbm,
          rows_v, z_v, si_v, di_v, pb_v, acc_sh, gsem):
        c = lax.axis_index("c")
        s = lax.axis_index("s")
        zrows = half_acc // NS
        orows = half_n // NS

        def zrow(r, _):
            def zcol(kk, _):
                z_v[r, pl.ds(kk * 16, 16)] = jnp.zeros((16,), jnp.float32)
                return 0

            lax.fori_loop(0, d // 16, zcol, 0)
            return 0

        lax.fori_loop(0, CH, zrow, 0)

        def zblk(i, _):
            pltpu.sync_copy(z_v, acc_sh.at[pl.ds(s * zrows + i * CH, CH)])
            return 0

        lax.fori_loop(0, zrows // CH, zblk, 0)

        pltpu.sync_copy(pbc_hbm, pb_v)
        base1 = pb_v[1, pl.ds(0, 16)][0]
        cnt0 = pb_v[2, pl.ds(0, 16)][0]
        cnt1 = pb_v[3, pl.ds(0, 16)][0]
        base_c = jnp.where(c == 0, 0, base1)
        cnt_c = jnp.where(c == 0, cnt0, cnt1)
        nch = (cnt_c + CH - 1) // CH
        q = nch // NS
        rem = nch - q * NS
        myn = q + jnp.where(s < rem, 1, 0)
        g0q = base_c // CH + s * q + jnp.minimum(s, rem)
        plsc.subcore_barrier()

        @pl.when(myn > 0)
        def _():
            pltpu.sync_copy(srcS_hbm.at[pl.ds(g0q * CH, CH)], si_v.at[0])
            pltpu.sync_copy(dstS_hbm.at[pl.ds(g0q * CH, CH)], di_v.at[0])
            pltpu.make_async_copy(
                ts_hbm.at[si_v.at[0]], rows_v.at[0], gsem.at[0]
            ).start()

            def body(i, _):
                p = lax.rem(i, 2)

                @pl.when(i + 1 < myn)
                def _():
                    pn = lax.rem(i + 1, 2)
                    g1 = (g0q + i + 1) * CH
                    pltpu.sync_copy(srcS_hbm.at[pl.ds(g1, CH)], si_v.at[pn])
                    pltpu.sync_copy(dstS_hbm.at[pl.ds(g1, CH)], di_v.at[pn])
                    pltpu.make_async_copy(
                        ts_hbm.at[si_v.at[pn]], rows_v.at[pn], gsem.at[pn]
                    ).start()

                pltpu.make_async_copy(
                    ts_hbm.at[si_v.at[p]], rows_v.at[p], gsem.at[p]
                ).wait()
                pltpu.sync_copy(rows_v.at[p], acc_sh.at[di_v.at[p]], add=True)
                return 0

            lax.fori_loop(0, myn, body, 0)

        plsc.subcore_barrier()
        pltpu.sync_copy(
            acc_sh.at[pl.ds(s * orows, orows)],
            out_hbm.at[pl.ds((c * (half_n // 8) + s * (orows // 8)) * 8,
                             orows)],
        )

    return k(ts, srcS, dstS, pbc)


def _tc_prep(deg2):
    acc_n = deg2.shape[1]

    def body(deg_ref, dinv_ref):
        dinv_ref[...] = lax.rsqrt(deg_ref[0:1, :] + deg_ref[1:2, :] + 1.0)

    return pl.pallas_call(
        body, out_shape=jax.ShapeDtypeStruct((1, acc_n), jnp.float32)
    )(deg2)


def _tc_first(x, W1, dinv, blk):
    n, d = x.shape

    def body(x_ref, w_ref, dv_ref, out_ref):
        out_ref[...] = (
            jnp.dot(x_ref[...], w_ref[...], preferred_element_type=jnp.float32)
            * dv_ref[...]
        )

    return pl.pallas_call(
        body,
        grid=(n // blk,),
        in_specs=[
            pl.BlockSpec((blk, d), lambda i: (i, 0)),
            pl.BlockSpec((d, d), lambda i: (0, 0)),
            pl.BlockSpec((blk, 1), lambda i: (i, 0)),
        ],
        out_specs=pl.BlockSpec((blk, d), lambda i: (i, 0)),
        out_shape=jax.ShapeDtypeStruct((n, d), jnp.float32),
    )(x, W1, dinv)


def _tc_layer(p, ts, dinv, b, W, blk):
    n, d = ts.shape

    def body(p_ref, ts_ref, dv_ref, b_ref, w_ref, out_ref):
        h = (p_ref[...] + ts_ref[...]) * dv_ref[...] + b_ref[...]
        h = jnp.maximum(h, 0.0)
        out_ref[...] = (
            jnp.dot(h, w_ref[...], preferred_element_type=jnp.float32) * dv_ref[...]
        )

    return pl.pallas_call(
        body,
        grid=(n // blk,),
        in_specs=[
            pl.BlockSpec((blk, d), lambda i: (i, 0)),
            pl.BlockSpec((blk, d), lambda i: (i, 0)),
            pl.BlockSpec((blk, 1), lambda i: (i, 0)),
            pl.BlockSpec((1, d), lambda i: (0, 0)),
            pl.BlockSpec((d, d), lambda i: (0, 0)),
        ],
        out_specs=pl.BlockSpec((blk, d), lambda i: (i, 0)),
        out_shape=jax.ShapeDtypeStruct((n, d), jnp.float32),
    )(p, ts, dinv, b, W)


def _tc_head(p, ts, dinv, b3, L1w, L1b, L2w, L2b, blk, n_real):
    n, d = ts.shape
    g_steps = n // blk

    def body(p_ref, ts_ref, dv_ref, b_ref, l1w_ref, l1b_ref, l2w_ref, l2b_ref,
             out_ref, acc_ref):
        i = pl.program_id(0)
        h = (p_ref[...] + ts_ref[...]) * dv_ref[...] + b_ref[...]
        h = jnp.maximum(h, 0.0)
        bsum = jnp.sum(h, axis=0, keepdims=True)

        @pl.when(i == 0)
        def _():
            acc_ref[...] = bsum

        @pl.when(i > 0)
        def _():
            acc_ref[...] = acc_ref[...] + bsum

        @pl.when(i == g_steps - 1)
        def _():
            g = acc_ref[...] * (1.0 / n_real)
            z = jnp.dot(g, l1w_ref[...], preferred_element_type=jnp.float32)
            z = jnp.maximum(z + l1b_ref[...], 0.0)
            o = jnp.dot(z, l2w_ref[...], preferred_element_type=jnp.float32)
            out_ref[...] = jax.nn.sigmoid(o + l2b_ref[...])

    return pl.pallas_call(
        body,
        grid=(g_steps,),
        in_specs=[
            pl.BlockSpec((blk, d), lambda i: (i, 0)),
            pl.BlockSpec((blk, d), lambda i: (i, 0)),
            pl.BlockSpec((blk, 1), lambda i: (i, 0)),
            pl.BlockSpec((1, d), lambda i: (0, 0)),
            pl.BlockSpec((d, d), lambda i: (0, 0)),
            pl.BlockSpec((1, d), lambda i: (0, 0)),
            pl.BlockSpec((d, 1), lambda i: (0, 0)),
            pl.BlockSpec((1, 1), lambda i: (0, 0)),
        ],
        out_specs=pl.BlockSpec((1, 1), lambda i: (0, 0)),
        out_shape=jax.ShapeDtypeStruct((1, 1), jnp.float32),
        scratch_shapes=[pltpu.VMEM((1, d), jnp.float32)],
    )(p, ts, dinv, b3, L1w, L1b, L2w, L2b)


def kernel(x, edge_index, W1, b1, W2, b2, W3, b3, L1w, L1b, L2w, L2b):
    n, d = x.shape
    e = edge_index.shape[1]
    blk = 2000

    ept = ((e + NW * CH - 1) // (NW * CH)) * CH
    cpt = ept // CH
    epad = NW * ept
    acc_n = ((n + NS * CH) // (NS * CH)) * NS * CH
    half_n = acc_n // 2
    dumpn = 1024
    half_acc = half_n + dumpn
    nbk = 2
    cs = 256
    nblk = e // cs
    sort_sz = e + NW * CH
    sort_rd = ((sort_sz + 256 + NS * 2048 - 1) // (NS * 2048)) * (NS * 2048)

    src = edge_index[0]
    dst = edge_index[1]
    pad = epad - e

    dst3 = dst.reshape(nblk, cs, 1)
    pos3, pbc = _tc_positions(dst3, half_n, nbk, 10)
    posf = pos3.reshape(e)
    padpos = jnp.arange(pad, dtype=jnp.int32)
    srcp = jnp.concatenate([src, jnp.zeros((pad,), jnp.int32)]).reshape(
        NW, cpt, CH)
    dstp = jnp.concatenate([dst, jnp.full((pad,), n, jnp.int32)]).reshape(
        NW, cpt, CH)
    posp = jnp.concatenate([posf, padpos]).reshape(NW, cpt, CH)
    deg2, pS, pD = _sc_sortdeg(srcp, dstp, posp, pbc, n, acc_n, half_n,
                               sort_rd, cpt)
    dinv = _tc_prep(deg2).reshape(acc_n, 1)
    srcS2, dstS2 = _tc_merge(pS.reshape(NC, sort_rd // 128, 128),
                             pD.reshape(NC, sort_rd // 128, 128),
                             pbc, half_n, dumpn, 256)
    srcS = srcS2.reshape(sort_rd)
    dstS = dstS2.reshape(sort_rd)

    ts = _tc_first(x, W1, dinv, blk)
    p = _sc_aggregate(ts, srcS, dstS, pbc, acc_n, half_n, half_acc)
    ts = _tc_layer(p, ts, dinv, b1.reshape(1, d), W2, blk)
    p = _sc_aggregate(ts, srcS, dstS, pbc, acc_n, half_n, half_acc)
    ts = _tc_layer(p, ts, dinv, b2.reshape(1, d), W3, blk)
    p = _sc_aggregate(ts, srcS, dstS, pbc, acc_n, half_n, half_acc)
    out = _tc_head(p, ts, dinv, b3.reshape(1, d), L1w, L1b.reshape(1, d),
                   L2w, L2b.reshape(1, 1), blk, n)
    return out.reshape(1)


# speedup vs baseline: 4.8431x; 1.9472x over previous
import functools

import jax
import jax.numpy as jnp
from jax import lax
from jax.experimental import pallas as pl
from jax.experimental.pallas import tpu as pltpu
from jax.experimental.pallas import tpu_sc as plsc

NC = 2
NS = 16
NW = NC * NS
CH = 128


def _mesh():
    return plsc.VectorSubcoreMesh(core_axis_name="c", subcore_axis_name="s")


def _sc_sortdeg(srcp, dstp, posp, pbc, n, acc_n, half_n, sort_rd, cpt):
    @functools.partial(
        pl.kernel,
        out_type=(
            jax.ShapeDtypeStruct((NC, acc_n), jnp.float32),
            jax.ShapeDtypeStruct((NC, sort_rd), jnp.int32),
            jax.ShapeDtypeStruct((NC, sort_rd), jnp.int32),
        ),
        mesh=_mesh(),
        scratch_types=[
            pltpu.VMEM((cpt, CH), jnp.int32),
            pltpu.VMEM((cpt, CH), jnp.int32),
            pltpu.VMEM((cpt, CH), jnp.int32),
            pltpu.VMEM((CH,), jnp.float32),
            pltpu.VMEM((2048,), jnp.int32),
            pltpu.VMEM((acc_n // NS,), jnp.float32),
            pltpu.VMEM((8, CH), jnp.int32),
            pltpu.VMEM_SHARED((acc_n,), jnp.float32),
            pltpu.VMEM_SHARED((sort_rd,), jnp.int32),
            pltpu.VMEM_SHARED((sort_rd,), jnp.int32),
        ],
    )
    def k(src_hbm, dst_hbm, pos_hbm, pbc_hbm, deg_hbm, outS_hbm, outD_hbm,
          si_v, di_v, po_v, ones_v, z_v, zf_v, pb_v, deg_sh, srcS_sh, dstS_sh):
        c = lax.axis_index("c")
        s = lax.axis_index("s")
        w = s * NC + c
        zpt = sort_rd // NS
        dslice = acc_n // NS

        def fo(i, _):
            ones_v[pl.ds(i * 16, 16)] = jnp.ones((16,), jnp.float32)
            return 0

        lax.fori_loop(0, CH // 16, fo, 0)

        def zf(i, _):
            z_v[pl.ds(i * 16, 16)] = jnp.zeros((16,), jnp.int32)
            return 0

        lax.fori_loop(0, 2048 // 16, zf, 0)

        def zff(i, _):
            zf_v[pl.ds(i * 16, 16)] = jnp.zeros((16,), jnp.float32)
            return 0

        lax.fori_loop(0, dslice // 16, zff, 0)

        def zs(i, _):
            pltpu.sync_copy(z_v, srcS_sh.at[pl.ds(s * zpt + i * 2048, 2048)])
            pltpu.sync_copy(z_v, dstS_sh.at[pl.ds(s * zpt + i * 2048, 2048)])
            return 0

        lax.fori_loop(0, zpt // 2048, zs, 0)
        pltpu.sync_copy(zf_v, deg_sh.at[pl.ds(s * dslice, dslice)])
        pltpu.sync_copy(src_hbm.at[w], si_v)
        pltpu.sync_copy(dst_hbm.at[w], di_v)
        pltpu.sync_copy(pos_hbm.at[w], po_v)
        pltpu.sync_copy(pbc_hbm, pb_v)
        base1 = pb_v[1, pl.ds(0, 16)][0]
        cnt1 = pb_v[3, pl.ds(0, 16)][0]

        def fix(g, _):
            j = g // (CH // 16)
            kk = lax.rem(g, CH // 16)
            vd = di_v[j, pl.ds(kk * 16, 16)]
            vp = po_v[j, pl.ds(kk * 16, 16)]
            add = (jnp.where(vd >= half_n, base1, 0)
                   + jnp.where(vd >= n, cnt1, 0))
            po_v[j, pl.ds(kk * 16, 16)] = vp + add
            return 0

        lax.fori_loop(0, cpt * (CH // 16), fix, 0)
        plsc.subcore_barrier()

        def body(j, _):
            pltpu.sync_copy(ones_v, deg_sh.at[di_v.at[j]], add=True)
            pltpu.sync_copy(si_v.at[j], srcS_sh.at[po_v.at[j]], add=True)
            pltpu.sync_copy(di_v.at[j], dstS_sh.at[po_v.at[j]], add=True)
            return 0

        lax.fori_loop(0, cpt, body, 0)
        plsc.subcore_barrier()
        pltpu.sync_copy(deg_sh.at[pl.ds(s * dslice, dslice)],
                        deg_hbm.at[c, pl.ds(s * dslice, dslice)])
        pltpu.sync_copy(srcS_sh.at[pl.ds(s * zpt, zpt)],
                        outS_hbm.at[c, pl.ds(s * zpt, zpt)])
        pltpu.sync_copy(dstS_sh.at[pl.ds(s * zpt, zpt)],
                        outD_hbm.at[c, pl.ds(s * zpt, zpt)])

    return k(srcp, dstp, posp, pbc)


def _tc_positions(dst2, half_n, e):
    rows, lanes = dst2.shape
    blkr = rows // 10
    steps = rows // blkr

    def body(dst_ref, pos_ref, pbc_ref, run_ref):
        i = pl.program_id(0)

        @pl.when(i == 0)
        def _():
            run_ref[...] = jnp.zeros((1, 1), jnp.float32)

        x = jnp.where(dst_ref[...] >= half_n, 1.0, 0.0)
        ku = jax.lax.broadcasted_iota(jnp.int32, (lanes, lanes), 0)
        lu = jax.lax.broadcasted_iota(jnp.int32, (lanes, lanes), 1)
        upper = jnp.where(ku <= lu, 1.0, 0.0)
        lcum = jnp.dot(x, upper, preferred_element_type=jnp.float32)
        t = lcum[:, lanes - 1:lanes]
        ri = jax.lax.broadcasted_iota(jnp.int32, (blkr, blkr), 0)
        ci = jax.lax.broadcasted_iota(jnp.int32, (blkr, blkr), 1)
        tril = jnp.where(ci < ri, 1.0, 0.0)
        off = jnp.dot(tril, t, preferred_element_type=jnp.float32)
        run = run_ref[...]
        incl = lcum + off + run
        excl = incl - x
        rio = jax.lax.broadcasted_iota(jnp.int32, (blkr, lanes), 0)
        lio = jax.lax.broadcasted_iota(jnp.int32, (blkr, lanes), 1)
        gi = ((i * blkr + rio) * lanes + lio).astype(jnp.float32)
        posl = jnp.where(x > 0.0, excl, gi - excl)
        pos_ref[...] = posl.astype(jnp.int32)
        total = jnp.sum(t, axis=0, keepdims=True)
        run_ref[...] = run + total

        @pl.when(i == steps - 1)
        def _():
            cnt1 = run + total
            cnt0 = e - cnt1
            base1 = jnp.floor((cnt0 + 127.0) * (1.0 / 128.0)) * 128.0
            z18 = jnp.zeros((1, 128), jnp.float32)
            pbc_ref[0:1, :] = z18.astype(jnp.int32)
            pbc_ref[1:2, :] = jnp.broadcast_to(base1, (1, 128)).astype(jnp.int32)
            pbc_ref[2:3, :] = jnp.broadcast_to(cnt0, (1, 128)).astype(jnp.int32)
            pbc_ref[3:4, :] = jnp.broadcast_to(cnt1, (1, 128)).astype(jnp.int32)
            pbc_ref[4:8, :] = jnp.zeros((4, 128), jnp.int32)

    pos2, pbc = pl.pallas_call(
        body,
        grid=(steps,),
        in_specs=[pl.BlockSpec((blkr, lanes), lambda i: (i, 0))],
        out_specs=[pl.BlockSpec((blkr, lanes), lambda i: (i, 0)),
                   pl.BlockSpec((8, 128), lambda i: (0, 0))],
        out_shape=[
            jax.ShapeDtypeStruct((rows, lanes), jnp.int32),
            jax.ShapeDtypeStruct((8, 128), jnp.int32),
        ],
        scratch_shapes=[pltpu.VMEM((1, 1), jnp.float32)],
    )(dst2)
    return pos2, pbc


def _tc_merge(pS, pD, pbc, half_n, dumpn, blkr):
    nc, rows, lanes = pS.shape
    steps = rows // blkr

    def body(s_ref, d_ref, pbc_ref, os_ref, od_ref):
        ib = pl.program_id(0)
        base1 = pbc_ref[1, 0]
        cnt0 = pbc_ref[2, 0]
        cnt1 = pbc_ref[3, 0]
        sm = jnp.maximum(s_ref[0], s_ref[1])
        dm = jnp.maximum(d_ref[0], d_ref[1])
        ri = jax.lax.broadcasted_iota(jnp.int32, (blkr, lanes), 0)
        li = jax.lax.broadcasted_iota(jnp.int32, (blkr, lanes), 1)
        gi = (ib * blkr + ri) * lanes + li
        reg1 = gi >= base1
        local = gi - jnp.where(reg1, base1, 0)
        dead = local >= jnp.where(reg1, cnt1, cnt0)
        dmr = dm - jnp.where(reg1, half_n, 0)
        od_ref[...] = jnp.where(dead, half_n + jnp.remainder(gi, dumpn), dmr)
        os_ref[...] = jnp.where(dead, 0, sm)

    del nc
    return pl.pallas_call(
        body,
        grid=(steps,),
        in_specs=[
            pl.BlockSpec((2, blkr, lanes), lambda i: (0, i, 0)),
            pl.BlockSpec((2, blkr, lanes), lambda i: (0, i, 0)),
            pl.BlockSpec((8, 128), lambda i: (0, 0)),
        ],
        out_specs=[pl.BlockSpec((blkr, lanes), lambda i: (i, 0)),
                   pl.BlockSpec((blkr, lanes), lambda i: (i, 0))],
        out_shape=[
            jax.ShapeDtypeStruct((rows, lanes), jnp.int32),
            jax.ShapeDtypeStruct((rows, lanes), jnp.int32),
        ],
    )(pS, pD, pbc)


def _sc_aggregate(ts, srcS, dstS, pbc, acc_n, half_n, half_acc):
    d = ts.shape[1]

    @functools.partial(
        pl.kernel,
        out_type=jax.ShapeDtypeStruct((acc_n, d), jnp.float32),
        mesh=_mesh(),
        scratch_types=[
            pltpu.VMEM((2, CH, d), jnp.float32),
            pltpu.VMEM((CH, d), jnp.float32),
            pltpu.VMEM((2, CH), jnp.int32),
            pltpu.VMEM((2, CH), jnp.int32),
            pltpu.VMEM((8, CH), jnp.int32),
            pltpu.VMEM_SHARED((half_acc, d), jnp.float32),
            pltpu.SemaphoreType.DMA((2,)),
        ],
    )
    def k(ts_hbm, srcS_hbm, dstS_hbm, pbc_hbm, out_hbm,
          rows_v, z_v, si_v, di_v, pb_v, acc_sh, gsem):
        c = lax.axis_index("c")
        s = lax.axis_index("s")
        zrows = half_acc // NS
        orows = half_n // NS

        def zrow(r, _):
            def zcol(kk, _):
                z_v[r, pl.ds(kk * 16, 16)] = jnp.zeros((16,), jnp.float32)
                return 0

            lax.fori_loop(0, d // 16, zcol, 0)
            return 0

        lax.fori_loop(0, CH, zrow, 0)

        def zblk(i, _):
            pltpu.sync_copy(z_v, acc_sh.at[pl.ds(s * zrows + i * CH, CH)])
            return 0

        lax.fori_loop(0, zrows // CH, zblk, 0)

        pltpu.sync_copy(pbc_hbm, pb_v)
        base1 = pb_v[1, pl.ds(0, 16)][0]
        cnt0 = pb_v[2, pl.ds(0, 16)][0]
        cnt1 = pb_v[3, pl.ds(0, 16)][0]
        base_c = jnp.where(c == 0, 0, base1)
        cnt_c = jnp.where(c == 0, cnt0, cnt1)
        nch = (cnt_c + CH - 1) // CH
        q = nch // NS
        rem = nch - q * NS
        myn = q + jnp.where(s < rem, 1, 0)
        g0q = base_c // CH + s * q + jnp.minimum(s, rem)
        plsc.subcore_barrier()

        @pl.when(myn > 0)
        def _():
            pltpu.sync_copy(srcS_hbm.at[pl.ds(g0q * CH, CH)], si_v.at[0])
            pltpu.sync_copy(dstS_hbm.at[pl.ds(g0q * CH, CH)], di_v.at[0])
            pltpu.make_async_copy(
                ts_hbm.at[si_v.at[0]], rows_v.at[0], gsem.at[0]
            ).start()

            def body(i, _):
                p = lax.rem(i, 2)

                @pl.when(i + 1 < myn)
                def _():
                    pn = lax.rem(i + 1, 2)
                    g1 = (g0q + i + 1) * CH
                    pltpu.sync_copy(srcS_hbm.at[pl.ds(g1, CH)], si_v.at[pn])
                    pltpu.sync_copy(dstS_hbm.at[pl.ds(g1, CH)], di_v.at[pn])
                    pltpu.make_async_copy(
                        ts_hbm.at[si_v.at[pn]], rows_v.at[pn], gsem.at[pn]
                    ).start()

                pltpu.make_async_copy(
                    ts_hbm.at[si_v.at[p]], rows_v.at[p], gsem.at[p]
                ).wait()
                pltpu.sync_copy(rows_v.at[p], acc_sh.at[di_v.at[p]], add=True)
                return 0

            lax.fori_loop(0, myn, body, 0)

        plsc.subcore_barrier()
        pltpu.sync_copy(
            acc_sh.at[pl.ds(s * orows, orows)],
            out_hbm.at[pl.ds((c * (half_n // 8) + s * (orows // 8)) * 8,
                             orows)],
        )

    return k(ts, srcS, dstS, pbc)


def _tc_prep(deg2):
    acc_n = deg2.shape[1]

    def body(deg_ref, dinv_ref):
        dinv_ref[...] = lax.rsqrt(deg_ref[0:1, :] + deg_ref[1:2, :] + 1.0)

    return pl.pallas_call(
        body, out_shape=jax.ShapeDtypeStruct((1, acc_n), jnp.float32)
    )(deg2)


def _tc_first(x, W1, dinv, blk):
    n, d = x.shape

    def body(x_ref, w_ref, dv_ref, out_ref):
        out_ref[...] = (
            jnp.dot(x_ref[...], w_ref[...], preferred_element_type=jnp.float32)
            * dv_ref[...]
        )

    return pl.pallas_call(
        body,
        grid=(n // blk,),
        in_specs=[
            pl.BlockSpec((blk, d), lambda i: (i, 0)),
            pl.BlockSpec((d, d), lambda i: (0, 0)),
            pl.BlockSpec((blk, 1), lambda i: (i, 0)),
        ],
        out_specs=pl.BlockSpec((blk, d), lambda i: (i, 0)),
        out_shape=jax.ShapeDtypeStruct((n, d), jnp.float32),
    )(x, W1, dinv)


def _tc_layer(p, ts, dinv, b, W, blk):
    n, d = ts.shape

    def body(p_ref, ts_ref, dv_ref, b_ref, w_ref, out_ref):
        h = (p_ref[...] + ts_ref[...]) * dv_ref[...] + b_ref[...]
        h = jnp.maximum(h, 0.0)
        out_ref[...] = (
            jnp.dot(h, w_ref[...], preferred_element_type=jnp.float32) * dv_ref[...]
        )

    return pl.pallas_call(
        body,
        grid=(n // blk,),
        in_specs=[
            pl.BlockSpec((blk, d), lambda i: (i, 0)),
            pl.BlockSpec((blk, d), lambda i: (i, 0)),
            pl.BlockSpec((blk, 1), lambda i: (i, 0)),
            pl.BlockSpec((1, d), lambda i: (0, 0)),
            pl.BlockSpec((d, d), lambda i: (0, 0)),
        ],
        out_specs=pl.BlockSpec((blk, d), lambda i: (i, 0)),
        out_shape=jax.ShapeDtypeStruct((n, d), jnp.float32),
    )(p, ts, dinv, b, W)


def _tc_head(p, ts, dinv, b3, L1w, L1b, L2w, L2b, blk, n_real):
    n, d = ts.shape
    g_steps = n // blk

    def body(p_ref, ts_ref, dv_ref, b_ref, l1w_ref, l1b_ref, l2w_ref, l2b_ref,
             out_ref, acc_ref):
        i = pl.program_id(0)
        h = (p_ref[...] + ts_ref[...]) * dv_ref[...] + b_ref[...]
        h = jnp.maximum(h, 0.0)
        bsum = jnp.sum(h, axis=0, keepdims=True)

        @pl.when(i == 0)
        def _():
            acc_ref[...] = bsum

        @pl.when(i > 0)
        def _():
            acc_ref[...] = acc_ref[...] + bsum

        @pl.when(i == g_steps - 1)
        def _():
            g = acc_ref[...] * (1.0 / n_real)
            z = jnp.dot(g, l1w_ref[...], preferred_element_type=jnp.float32)
            z = jnp.maximum(z + l1b_ref[...], 0.0)
            o = jnp.dot(z, l2w_ref[...], preferred_element_type=jnp.float32)
            out_ref[...] = jax.nn.sigmoid(o + l2b_ref[...])

    return pl.pallas_call(
        body,
        grid=(g_steps,),
        in_specs=[
            pl.BlockSpec((blk, d), lambda i: (i, 0)),
            pl.BlockSpec((blk, d), lambda i: (i, 0)),
            pl.BlockSpec((blk, 1), lambda i: (i, 0)),
            pl.BlockSpec((1, d), lambda i: (0, 0)),
            pl.BlockSpec((d, d), lambda i: (0, 0)),
            pl.BlockSpec((1, d), lambda i: (0, 0)),
            pl.BlockSpec((d, 1), lambda i: (0, 0)),
            pl.BlockSpec((1, 1), lambda i: (0, 0)),
        ],
        out_specs=pl.BlockSpec((1, 1), lambda i: (0, 0)),
        out_shape=jax.ShapeDtypeStruct((1, 1), jnp.float32),
        scratch_shapes=[pltpu.VMEM((1, d), jnp.float32)],
    )(p, ts, dinv, b3, L1w, L1b, L2w, L2b)


def kernel(x, edge_index, W1, b1, W2, b2, W3, b3, L1w, L1b, L2w, L2b):
    n, d = x.shape
    e = edge_index.shape[1]
    blk = 2000

    ept = ((e + NW * CH - 1) // (NW * CH)) * CH
    cpt = ept // CH
    epad = NW * ept
    acc_n = ((n + NS * CH) // (NS * CH)) * NS * CH
    half_n = acc_n // 2
    dumpn = 1024
    half_acc = half_n + dumpn
    nbk = 2
    cs = 256
    nblk = e // cs
    sort_sz = e + NW * CH
    sort_rd = ((sort_sz + 256 + NS * 2048 - 1) // (NS * 2048)) * (NS * 2048)

    src = edge_index[0]
    dst = edge_index[1]
    pad = epad - e

    prows = ((e // 128 + 79) // 80) * 80
    dst2 = jnp.concatenate(
        [dst, jnp.zeros((prows * 128 - e,), jnp.int32)]).reshape(prows, 128)
    pos2, pbc = _tc_positions(dst2, half_n, e)
    posf = pos2.reshape(prows * 128)[:e]
    padpos = jnp.arange(pad, dtype=jnp.int32)
    srcp = jnp.concatenate([src, jnp.zeros((pad,), jnp.int32)]).reshape(
        NW, cpt, CH)
    dstp = jnp.concatenate([dst, jnp.full((pad,), n, jnp.int32)]).reshape(
        NW, cpt, CH)
    posp = jnp.concatenate([posf, padpos]).reshape(NW, cpt, CH)
    deg2, pS, pD = _sc_sortdeg(srcp, dstp, posp, pbc, n, acc_n, half_n,
                               sort_rd, cpt)
    dinv = _tc_prep(deg2).reshape(acc_n, 1)
    srcS2, dstS2 = _tc_merge(pS.reshape(NC, sort_rd // 128, 128),
                             pD.reshape(NC, sort_rd // 128, 128),
                             pbc, half_n, dumpn, 256)
    srcS = srcS2.reshape(sort_rd)
    dstS = dstS2.reshape(sort_rd)

    ts = _tc_first(x, W1, dinv, blk)
    p = _sc_aggregate(ts, srcS, dstS, pbc, acc_n, half_n, half_acc)
    ts = _tc_layer(p, ts, dinv, b1.reshape(1, d), W2, blk)
    p = _sc_aggregate(ts, srcS, dstS, pbc, acc_n, half_n, half_acc)
    ts = _tc_layer(p, ts, dinv, b2.reshape(1, d), W3, blk)
    p = _sc_aggregate(ts, srcS, dstS, pbc, acc_n, half_n, half_acc)
    out = _tc_head(p, ts, dinv, b3.reshape(1, d), L1w, L1b.reshape(1, d),
                   L2w, L2b.reshape(1, 1), blk, n)
    return out.reshape(1)
